# baseline (speedup 1.0000x reference)
import functools

import jax
import jax.numpy as jnp
from jax import lax
from jax.experimental import pallas as pl
from jax.experimental.pallas import tpu as pltpu
from jax.experimental.pallas import tpu_sc as plsc

_NC = 2
_NS = 16
_NW = _NC * _NS
_D = 64
_CH = 64
_SUB = 16
_NSUB = _CH // _SUB
_NBS = 4
_M = 5


def _sc_gather_pool(item_ids, gids, dids, wids, item_tab, g_tab, d_tab, w_tab):
    BL = item_ids.shape[0]
    tok_per_w = BL // _NW
    nchunk = tok_per_w // _CH

    mesh = plsc.VectorSubcoreMesh(core_axis_name="c", subcore_axis_name="s")

    @functools.partial(
        pl.kernel,
        out_type=[jax.ShapeDtypeStruct((BL, _D), jnp.float32) for _ in range(4)],
        mesh=mesh,
        compiler_params=pltpu.CompilerParams(use_tc_tiling_on_sc=False),
        scratch_types=dict(
            iid=[pltpu.VMEM((_CH,), jnp.int32) for _ in range(2)],
            gid=[pltpu.VMEM((_CH * _M,), jnp.int32) for _ in range(2)],
            did=[pltpu.VMEM((_CH * _M,), jnp.int32) for _ in range(2)],
            wid=[pltpu.VMEM((_CH * _M,), jnp.int32) for _ in range(2)],
            gbuf=[pltpu.VMEM((_SUB * _M, _D), jnp.float32) for _ in range(_NBS)],
            dbuf=[pltpu.VMEM((_SUB * _M, _D), jnp.float32) for _ in range(_NBS)],
            wbuf=[pltpu.VMEM((_SUB * _M, _D), jnp.float32) for _ in range(_NBS)],
            iacc=[pltpu.VMEM((_CH, _D), jnp.float32) for _ in range(2)],
            gacc=[pltpu.VMEM((_CH, _D), jnp.float32) for _ in range(2)],
            dacc=[pltpu.VMEM((_CH, _D), jnp.float32) for _ in range(2)],
            wacc=[pltpu.VMEM((_CH, _D), jnp.float32) for _ in range(2)],
            sem_ids=[pltpu.SemaphoreType.DMA for _ in range(2)],
            sem_g=[pltpu.SemaphoreType.DMA for _ in range(_NBS)],
            sem_out=[pltpu.SemaphoreType.DMA for _ in range(2)],
        ),
    )
    def body(iid_h, gid_h, did_h, wid_h, itab_h, gtab_h, dtab_h, wtab_h,
             item_out, g_out, d_out, w_out, *, iid, gid, did, wid,
             gbuf, dbuf, wbuf, iacc, gacc, dacc, wacc,
             sem_ids, sem_g, sem_out):
        wkr = lax.axis_index("s") * _NC + lax.axis_index("c")
        w_base = wkr * tok_per_w

        def ids_copies(c, cs):
            base = w_base + c * _CH
            return (
                pltpu.make_async_copy(iid_h.at[pl.ds(base, _CH)], iid[cs], sem_ids[cs]),
                pltpu.make_async_copy(gid_h.at[pl.ds(base * _M, _CH * _M)], gid[cs], sem_ids[cs]),
                pltpu.make_async_copy(did_h.at[pl.ds(base * _M, _CH * _M)], did[cs], sem_ids[cs]),
                pltpu.make_async_copy(wid_h.at[pl.ds(base * _M, _CH * _M)], wid[cs], sem_ids[cs]),
            )

        def ids_start(c, cs):
            for cp in ids_copies(c, cs):
                cp.start()

        def ids_wait(c, cs):
            for cp in ids_copies(c, cs):
                cp.wait()

        def gather_copies(s, cs, bs):
            tok0 = s * _SUB
            return (
                pltpu.make_async_copy(
                    itab_h.at[iid[cs].at[pl.ds(tok0, _SUB)]],
                    iacc[cs].at[pl.ds(tok0, _SUB)], sem_g[bs]),
                pltpu.make_async_copy(
                    gtab_h.at[gid[cs].at[pl.ds(tok0 * _M, _SUB * _M)]],
                    gbuf[bs], sem_g[bs]),
                pltpu.make_async_copy(
                    dtab_h.at[did[cs].at[pl.ds(tok0 * _M, _SUB * _M)]],
                    dbuf[bs], sem_g[bs]),
                pltpu.make_async_copy(
                    wtab_h.at[wid[cs].at[pl.ds(tok0 * _M, _SUB * _M)]],
                    wbuf[bs], sem_g[bs]),
            )

        def gather_start(s, cs, bs):
            for cp in gather_copies(s, cs, bs):
                cp.start()

        def gather_wait(s, cs, bs):
            for cp in gather_copies(s, cs, bs):
                cp.wait()

        def pool(s, cs, bs):
            tok0 = s * _SUB

            def tree5(buf, r, sl):
                return ((buf[r, sl] + buf[r + 1, sl])
                        + (buf[r + 2, sl] + buf[r + 3, sl])) + buf[r + 4, sl]

            @plsc.parallel_loop(0, _SUB, 1, unroll=4)
            def tok(t):
                r = t * _M
                for ch in range(_D // 16):
                    sl = pl.ds(ch * 16, 16)
                    gacc[cs][tok0 + t, sl] = tree5(gbuf[bs], r, sl)
                    dacc[cs][tok0 + t, sl] = tree5(dbuf[bs], r, sl)
                    wacc[cs][tok0 + t, sl] = tree5(wbuf[bs], r, sl)

        def out_copies(c, cs):
            sl = pl.ds(w_base + c * _CH, _CH)
            return (
                pltpu.make_async_copy(iacc[cs], item_out.at[sl], sem_out[cs]),
                pltpu.make_async_copy(gacc[cs], g_out.at[sl], sem_out[cs]),
                pltpu.make_async_copy(dacc[cs], d_out.at[sl], sem_out[cs]),
                pltpu.make_async_copy(wacc[cs], w_out.at[sl], sem_out[cs]),
            )

        def out_start(c, cs):
            for cp in out_copies(c, cs):
                cp.start()

        def out_wait(c, cs):
            for cp in out_copies(c, cs):
                cp.wait()

        ids_start(0, 0)
        ids_start(1, 1)
        ids_wait(0, 0)
        gather_start(0, 0, 0)
        gather_start(1, 0, 1)

        def chunk_pair(p, carry):
            c0 = 2 * p
            for cs in range(2):
                c = c0 + cs
                for s in range(_NSUB):
                    bs = s % _NBS
                    gather_wait(s, cs, bs)
                    if s + 2 < _NSUB:
                        gather_start(s + 2, cs, bs_next := (s + 2) % _NBS)
                    else:
                        s2 = s + 2 - _NSUB
                        ncs = 1 - cs
                        if cs == 0:
                            if s2 == 0:
                                ids_wait(c + 1, ncs)

                                @pl.when(c + 1 >= 2)
                                def _():
                                    out_wait(c - 1, ncs)
                            gather_start(s2, ncs, s2 % _NBS)
                        else:
                            @pl.when(c + 1 < nchunk)
                            def _():
                                if s2 == 0:
                                    ids_wait(c + 1, ncs)
                                    out_wait(c - 1, ncs)
                                gather_start(s2, ncs, s2 % _NBS)
                    pool(s, cs, bs)
                out_start(c, cs)

                @pl.when(c + 2 < nchunk)
                def _():
                    ids_start(c + 2, cs)
            return carry

        lax.fori_loop(0, nchunk // 2, chunk_pair, 0)
        out_wait(nchunk - 2, 0)
        out_wait(nchunk - 1, 1)

    return body(item_ids, gids, dids, wids, item_tab, g_tab, d_tab, w_tab)


def _tc_project(item_e, g_sum, d_sum, w_sum, w_eff, b2):
    BL = item_e.shape[0]
    BT = 1024
    grid = (BL // BT,)

    def mm(ie, g, d, w, wr, br, o):
        x = jnp.concatenate([ie[...], g[...], d[...], w[...]], axis=1)
        o[...] = jnp.dot(x, wr[...], preferred_element_type=jnp.float32) + br[...]

    return pl.pallas_call(
        mm,
        grid=grid,
        in_specs=[
            pl.BlockSpec((BT, _D), lambda i: (i, 0)),
            pl.BlockSpec((BT, _D), lambda i: (i, 0)),
            pl.BlockSpec((BT, _D), lambda i: (i, 0)),
            pl.BlockSpec((BT, _D), lambda i: (i, 0)),
            pl.BlockSpec((4 * _D, _D), lambda i: (0, 0)),
            pl.BlockSpec((1, _D), lambda i: (0, 0)),
        ],
        out_specs=pl.BlockSpec((BT, _D), lambda i: (i, 0)),
        out_shape=jax.ShapeDtypeStruct((BL, _D), jnp.float32),
    )(item_e, g_sum, d_sum, w_sum, w_eff, b2)


def kernel(item_ids, genre_ids, director_ids, writer_ids, item_table,
           genre_table, director_table, writer_table, W, b):
    B, L = item_ids.shape
    BL = B * L
    M = genre_ids.shape[-1]

    ii = item_ids.reshape(BL).astype(jnp.int32)
    gi = genre_ids.reshape(BL * M).astype(jnp.int32)
    di = director_ids.reshape(BL * M).astype(jnp.int32)
    wi = writer_ids.reshape(BL * M).astype(jnp.int32)

    item_e, g_sum, d_sum, w_sum = _sc_gather_pool(
        ii, gi, di, wi, item_table, genre_table, director_table, writer_table)

    w_eff = jnp.concatenate([W[:_D], W[_D:] * (1.0 / M)], axis=0)
    out = _tc_project(item_e, g_sum, d_sum, w_sum, w_eff, b.reshape(1, _D))
    return out.reshape(B, L, _D)


# speedup vs baseline: 1.4786x; 1.4786x over previous
import functools

import jax
import jax.numpy as jnp
from jax import lax
from jax.experimental import pallas as pl
from jax.experimental.pallas import tpu as pltpu
from jax.experimental.pallas import tpu_sc as plsc

_NC = 2
_NS = 16
_NW = _NC * _NS
_D = 64
_CH = 64
_SUB = 16
_NSUB = _CH // _SUB
_NBS = 4
_M = 5


def _sc_gather_pool(item_ids, gids, dids, wids, item_tab, g_tab, d_tab, w_tab):
    BL = item_ids.shape[0]
    tok_per_w = BL // _NW
    nchunk = tok_per_w // _CH

    mesh = plsc.VectorSubcoreMesh(core_axis_name="c", subcore_axis_name="s")

    @functools.partial(
        pl.kernel,
        out_type=[jax.ShapeDtypeStruct((BL * _D,), jnp.float32) for _ in range(4)],
        mesh=mesh,
        compiler_params=pltpu.CompilerParams(use_tc_tiling_on_sc=False),
        scratch_types=dict(
            iid=[pltpu.VMEM((_CH,), jnp.int32) for _ in range(2)],
            gid=[pltpu.VMEM((_M, _CH), jnp.int32) for _ in range(2)],
            did=[pltpu.VMEM((_M, _CH), jnp.int32) for _ in range(2)],
            wid=[pltpu.VMEM((_M, _CH), jnp.int32) for _ in range(2)],
            ibuf=[pltpu.VMEM((_SUB, _D), jnp.float32) for _ in range(_NBS)],
            gbuf=[pltpu.VMEM((_SUB * _M, _D), jnp.float32) for _ in range(_NBS)],
            dbuf=[pltpu.VMEM((_SUB * _M, _D), jnp.float32) for _ in range(_NBS)],
            wbuf=[pltpu.VMEM((_SUB * _M, _D), jnp.float32) for _ in range(_NBS)],
            iacc=[pltpu.VMEM((_CH * _D,), jnp.float32) for _ in range(2)],
            gacc=[pltpu.VMEM((_CH * _D,), jnp.float32) for _ in range(2)],
            dacc=[pltpu.VMEM((_CH * _D,), jnp.float32) for _ in range(2)],
            wacc=[pltpu.VMEM((_CH * _D,), jnp.float32) for _ in range(2)],
            sem_ids=[pltpu.SemaphoreType.DMA for _ in range(2)],
            sem_g=[pltpu.SemaphoreType.DMA for _ in range(_NBS)],
            sem_out=[pltpu.SemaphoreType.DMA for _ in range(2)],
        ),
    )
    def body(iid_h, gid_h, did_h, wid_h, itab_h, gtab_h, dtab_h, wtab_h,
             item_out, g_out, d_out, w_out, *, iid, gid, did, wid,
             ibuf, gbuf, dbuf, wbuf, iacc, gacc, dacc, wacc,
             sem_ids, sem_g, sem_out):
        wkr = lax.axis_index("s") * _NC + lax.axis_index("c")
        w_base = wkr * tok_per_w

        def ids_copies(c, cs):
            base = w_base + c * _CH
            sl = pl.ds(base, _CH)
            return (
                pltpu.make_async_copy(iid_h.at[sl], iid[cs], sem_ids[cs]),
                pltpu.make_async_copy(gid_h.at[:, sl], gid[cs], sem_ids[cs]),
                pltpu.make_async_copy(did_h.at[:, sl], did[cs], sem_ids[cs]),
                pltpu.make_async_copy(wid_h.at[:, sl], wid[cs], sem_ids[cs]),
            )

        def ids_start(c, cs):
            for cp in ids_copies(c, cs):
                cp.start()

        def ids_wait(c, cs):
            for cp in ids_copies(c, cs):
                cp.wait()

        def gather_copies(s, cs, bs):
            tok0 = s * _SUB
            sl = pl.ds(tok0, _SUB)
            cps = [pltpu.make_async_copy(
                itab_h.at[iid[cs].at[sl]], ibuf[bs], sem_g[bs])]
            for tab, idx, buf in ((gtab_h, gid, gbuf), (dtab_h, did, dbuf),
                                  (wtab_h, wid, wbuf)):
                for k in range(_M):
                    cps.append(pltpu.make_async_copy(
                        tab.at[idx[cs].at[k, sl]],
                        buf[bs].at[pl.ds(k * _SUB, _SUB)], sem_g[bs]))
            return cps

        def gather_start(s, cs, bs):
            for cp in gather_copies(s, cs, bs):
                cp.start()

        def gather_wait(s, cs, bs):
            for cp in gather_copies(s, cs, bs):
                cp.wait()

        def pool(s, cs, bs):
            tok0 = s * _SUB

            def tree5(buf, t, sl):
                return ((buf[t, sl] + buf[_SUB + t, sl])
                        + (buf[2 * _SUB + t, sl] + buf[3 * _SUB + t, sl])
                        ) + buf[4 * _SUB + t, sl]

            @plsc.parallel_loop(0, _SUB, 1, unroll=4)
            def tok(t):
                o = (tok0 + t) * _D
                for ch in range(_D // 16):
                    sl = pl.ds(ch * 16, 16)
                    so = pl.ds(o + ch * 16, 16)
                    iacc[cs][so] = ibuf[bs][t, sl]
                    gacc[cs][so] = tree5(gbuf[bs], t, sl)
                    dacc[cs][so] = tree5(dbuf[bs], t, sl)
                    wacc[cs][so] = tree5(wbuf[bs], t, sl)

        def out_copies(c, cs):
            sl = pl.ds((w_base + c * _CH) * _D, _CH * _D)
            return (
                pltpu.make_async_copy(iacc[cs], item_out.at[sl], sem_out[cs]),
                pltpu.make_async_copy(gacc[cs], g_out.at[sl], sem_out[cs]),
                pltpu.make_async_copy(dacc[cs], d_out.at[sl], sem_out[cs]),
                pltpu.make_async_copy(wacc[cs], w_out.at[sl], sem_out[cs]),
            )

        def out_start(c, cs):
            for cp in out_copies(c, cs):
                cp.start()

        def out_wait(c, cs):
            for cp in out_copies(c, cs):
                cp.wait()

        ids_start(0, 0)
        ids_start(1, 1)
        ids_wait(0, 0)
        gather_start(0, 0, 0)
        gather_start(1, 0, 1)

        def chunk_pair(p, carry):
            c0 = 2 * p
            for cs in range(2):
                c = c0 + cs
                for s in range(_NSUB):
                    bs = s % _NBS
                    gather_wait(s, cs, bs)
                    if s + 2 < _NSUB:
                        gather_start(s + 2, cs, (s + 2) % _NBS)
                    else:
                        s2 = s + 2 - _NSUB
                        ncs = 1 - cs
                        if cs == 0:
                            if s2 == 0:
                                ids_wait(c + 1, ncs)

                                @pl.when(c + 1 >= 2)
                                def _():
                                    out_wait(c - 1, ncs)
                            gather_start(s2, ncs, s2 % _NBS)
                        else:
                            @pl.when(c + 1 < nchunk)
                            def _():
                                if s2 == 0:
                                    ids_wait(c + 1, ncs)
                                    out_wait(c - 1, ncs)
                                gather_start(s2, ncs, s2 % _NBS)
                    pool(s, cs, bs)
                out_start(c, cs)

                @pl.when(c + 2 < nchunk)
                def _():
                    ids_start(c + 2, cs)
            return carry

        lax.fori_loop(0, nchunk // 2, chunk_pair, 0)
        out_wait(nchunk - 2, 0)
        out_wait(nchunk - 1, 1)

    return body(item_ids, gids, dids, wids, item_tab, g_tab, d_tab, w_tab)


def _tc_project(item_e, g_sum, d_sum, w_sum, w2, b2):
    BLD = item_e.shape[0]
    NP = BLD // (2 * _D)
    BT2 = 512
    grid = (NP // BT2,)

    def mm(i1, g1, d1, w1, wr, br, o):
        x = jnp.concatenate([
            i1[...].reshape(BT2, 2 * _D),
            g1[...].reshape(BT2, 2 * _D),
            d1[...].reshape(BT2, 2 * _D),
            w1[...].reshape(BT2, 2 * _D),
        ], axis=1)
        o[...] = jnp.dot(x, wr[...], preferred_element_type=jnp.float32) + br[...]

    return pl.pallas_call(
        mm,
        grid=grid,
        in_specs=[
            pl.BlockSpec((BT2 * 2 * _D,), lambda i: (i,)),
            pl.BlockSpec((BT2 * 2 * _D,), lambda i: (i,)),
            pl.BlockSpec((BT2 * 2 * _D,), lambda i: (i,)),
            pl.BlockSpec((BT2 * 2 * _D,), lambda i: (i,)),
            pl.BlockSpec((8 * _D, 2 * _D), lambda i: (0, 0)),
            pl.BlockSpec((1, 2 * _D), lambda i: (0, 0)),
        ],
        out_specs=pl.BlockSpec((BT2, 2 * _D), lambda i: (i, 0)),
        out_shape=jax.ShapeDtypeStruct((NP, 2 * _D), jnp.float32),
    )(item_e, g_sum, d_sum, w_sum, w2, b2)


def kernel(item_ids, genre_ids, director_ids, writer_ids, item_table,
           genre_table, director_table, writer_table, W, b):
    B, L = item_ids.shape
    BL = B * L
    M = genre_ids.shape[-1]

    ii = item_ids.reshape(BL).astype(jnp.int32)
    gi = genre_ids.reshape(BL, M).T.astype(jnp.int32)
    di = director_ids.reshape(BL, M).T.astype(jnp.int32)
    wi = writer_ids.reshape(BL, M).T.astype(jnp.int32)

    item_e, g_sum, d_sum, w_sum = _sc_gather_pool(
        ii, gi, di, wi, item_table, genre_table, director_table, writer_table)

    z = jnp.zeros((_D, _D), jnp.float32)

    def dbl(wk):
        return jnp.concatenate([
            jnp.concatenate([wk, z], axis=1),
            jnp.concatenate([z, wk], axis=1),
        ], axis=0)

    inv_m = 1.0 / M
    w2 = jnp.concatenate([
        dbl(W[:_D]),
        dbl(W[_D:2 * _D] * inv_m),
        dbl(W[2 * _D:3 * _D] * inv_m),
        dbl(W[3 * _D:] * inv_m),
    ], axis=0)
    b2 = jnp.concatenate([b, b]).reshape(1, 2 * _D)

    out2 = _tc_project(item_e, g_sum, d_sum, w_sum, w2, b2)
    return out2.reshape(B, L, _D)
